# Initial kernel scaffold; baseline (speedup 1.0000x reference)
import jax
import jax.numpy as jnp
from jax.experimental import pallas as pl


def kernel(input_atom, input_bond, atom_graph, bond_graph, num_nbs, node_mask, extra, W_af, W_na, W_nb, W_sa, W_u2, b_u2, W_u1, b_u1):
    raise NotImplementedError("write your pallas kernel here")


# speedup vs baseline: 22.9311x; 22.9311x over previous
import functools

import jax
import jax.numpy as jnp
from jax import lax
from jax.experimental import pallas as pl
from jax.experimental.pallas import tpu as pltpu
from jax.experimental.pallas import tpu_sc as plsc

_H = 128
_MAX_NB = 10
_DEPTH = 3
_B, _N, _NBOND = 128, 200, 400
_RA = _B * _N
_RB = _B * _NBOND
_NW = 32
_PW = _RA // _NW
_G = 8
_NCH = _PW // _G
_IPC = _G * _MAX_NB
_HC = _H // 16
_NFP = 5


_BRA = 1024
_BRB = 1024


def _rows(br):
    return pl.BlockSpec((br, _H), lambda i: (i, 0))


def _full(shape):
    return pl.BlockSpec(shape, lambda i: (0,) * len(shape))


def _tc_atoms0_body(xa, waf, wu2a, bu2, af_o, t_o):
    af = jnp.dot(xa[...], waf[...], preferred_element_type=jnp.float32)
    af_o[...] = af
    t_o[...] = jnp.dot(af, wu2a[...], preferred_element_type=jnp.float32) + bu2[...]


def _tc_bonds_body(xb, wnb, wu2b, bnb_o, bu2_o):
    x = xb[...]
    bnb_o[...] = jnp.dot(x, wnb[...], preferred_element_type=jnp.float32)
    bu2_o[...] = jnp.dot(x, wu2b[...], preferred_element_type=jnp.float32)


def _tc_update_body(full_tables, af, nei, w1a, w1b, b1, wu2a, bu2, *rest):
    if full_tables:
        wna, wsa, af_o, t_o, na_o, sa_o = rest
    else:
        af_o, t_o = rest
    afn = jnp.dot(af[...], w1a[...], preferred_element_type=jnp.float32)
    afn = afn + jnp.dot(nei[...], w1b[...], preferred_element_type=jnp.float32)
    afn = jnp.maximum(afn + b1[...], 0.0)
    af_o[...] = afn
    t_o[...] = jnp.dot(afn, wu2a[...], preferred_element_type=jnp.float32) + bu2[...]
    if full_tables:
        na_o[...] = jnp.dot(afn, wna[...], preferred_element_type=jnp.float32)
        sa_o[...] = jnp.dot(afn, wsa[...], preferred_element_type=jnp.float32)


def _tc_final_body(af, nei, w1a, w1b, b1, fnei, sa, af_o, k_o):
    afn = jnp.dot(af[...], w1a[...], preferred_element_type=jnp.float32)
    afn = afn + jnp.dot(nei[...], w1b[...], preferred_element_type=jnp.float32)
    af_o[...] = jnp.maximum(afn + b1[...], 0.0)
    k_o[...] = fnei[...] * sa[...]


def _tc_atoms0(xa, waf, wu2a, bu2):
    sds = jax.ShapeDtypeStruct((_RA, _H), jnp.float32)
    return pl.pallas_call(
        _tc_atoms0_body,
        grid=(_RA // _BRA,),
        in_specs=[pl.BlockSpec((_BRA, _H), lambda i: (i, 0)),
                  _full((_H, _H)), _full((_H, _H)), _full((1, _H))],
        out_specs=[_rows(_BRA), _rows(_BRA)],
        out_shape=[sds, sds],
    )(xa, waf, wu2a, bu2)


def _tc_bonds(xb, wnb, wu2b):
    sds = jax.ShapeDtypeStruct((_RB, _H), jnp.float32)
    return pl.pallas_call(
        _tc_bonds_body,
        grid=(_RB // _BRB,),
        in_specs=[pl.BlockSpec((_BRB, _H), lambda i: (i, 0)),
                  _full((_H, _H)), _full((_H, _H))],
        out_specs=[_rows(_BRB), _rows(_BRB)],
        out_shape=[sds, sds],
    )(xb, wnb, wu2b)


def _tc_update(full_tables, af, nei, w1a, w1b, b1, wu2a, bu2, *weights):
    sds = jax.ShapeDtypeStruct((_RA, _H), jnp.float32)
    n_out = 4 if full_tables else 2
    return pl.pallas_call(
        functools.partial(_tc_update_body, full_tables),
        grid=(_RA // _BRA,),
        in_specs=[_rows(_BRA), _rows(_BRA), _full((_H, _H)), _full((_H, _H)),
                  _full((1, _H)), _full((_H, _H)), _full((1, _H))]
                 + [_full((_H, _H))] * len(weights),
        out_specs=[_rows(_BRA)] * n_out,
        out_shape=[sds] * n_out,
    )(af, nei, w1a, w1b, b1, wu2a, bu2, *weights)


def _tc_final(af, nei, w1a, w1b, b1, fnei, sa):
    sds = jax.ShapeDtypeStruct((_RA, _H), jnp.float32)
    return pl.pallas_call(
        _tc_final_body,
        grid=(_RA // _BRA,),
        in_specs=[_rows(_BRA), _rows(_BRA), _full((_H, _H)), _full((_H, _H)),
                  _full((1, _H)), _rows(_BRA), _rows(_BRA)],
        out_specs=[_rows(_BRA), _rows(_BRA)],
        out_shape=[sds, sds],
    )(af, nei, w1a, w1b, b1, fnei, sa)


def _sc_gather_reduce(last_depth):
    f32 = jnp.float32
    info = plsc.get_sparse_core_info()
    nc = info.num_cores

    n_out = 2 if last_depth else 1
    out_type = [jax.ShapeDtypeStruct((_RA, _H), f32)] * n_out
    n_tab = 4 if last_depth else 2
    scratch = (
        [pltpu.VMEM((_NCH * _IPC,), jnp.int32)] * 2
        + [pltpu.VMEM((_NCH * _IPC // _NFP,), jnp.int32)]
        + [pltpu.VMEM((_NCH, 16), jnp.int32)]
        + [pltpu.VMEM((2, _IPC, _H), f32)] * n_tab
        + [pltpu.VMEM((_G, _H), f32)] * n_out
        + ([pltpu.VMEM((_NCH, 16), f32)] if last_depth else [])
        + [pltpu.SemaphoreType.DMA((2,))] * n_tab
    )

    def body(*refs):
        if last_depth:
            (ta, tb, tna, tnb, ag0, ag1, bg0, bg1, nnbh, nmh, nei_o, fnei_o,
             ia, ib, tmp, nnb, ra, rb, rna, rnb, stg, stg2, nm,
             sem_a, sem_b, sem_na, sem_nb) = refs
        else:
            (ta, tb, ag0, ag1, bg0, bg1, nnbh, nei_o,
             ia, ib, tmp, nnb, ra, rb, stg, sem_a, sem_b) = refs

        wid = lax.axis_index("s") * nc + lax.axis_index("c")

        n_idx = _NCH * _IPC
        npp = n_idx // _NFP

        def _flatten(dst, src0, src1, mul):
            pltpu.sync_copy(src0.at[pl.ds(wid * n_idx, n_idx)], dst)
            for q in range(_NFP):
                pltpu.sync_copy(src1.at[pl.ds(wid * n_idx + q * npp, npp)], tmp)

                def f(t, carry):
                    sd = pl.ds(q * npp + t * 16, 16)
                    st = pl.ds(t * 16, 16)
                    dst[sd] = dst[sd] * mul + tmp[st]
                    return carry
                lax.fori_loop(0, npp // 16, f, 0)

        _flatten(ia, ag0, ag1, _N)
        _flatten(ib, bg0, bg1, _NBOND)
        pltpu.sync_copy(nnbh.at[wid], nnb)
        if last_depth:
            pltpu.sync_copy(nmh.at[wid], nm)

        def _copies(c, p):
            sa = ia.at[pl.ds(c * _IPC, _IPC)]
            sb = ib.at[pl.ds(c * _IPC, _IPC)]
            cps = [pltpu.make_async_copy(ta.at[sa], ra.at[p], sem_a.at[p]),
                   pltpu.make_async_copy(tb.at[sb], rb.at[p], sem_b.at[p])]
            if last_depth:
                cps += [pltpu.make_async_copy(tna.at[sa], rna.at[p], sem_na.at[p]),
                        pltpu.make_async_copy(tnb.at[sb], rnb.at[p], sem_nb.at[p])]
            return cps

        def _start(c, p):
            for cp in _copies(c, p):
                cp.start()

        def _wait(p):
            for cp in _copies(0, p):
                cp.wait()

        zeros = tuple(jnp.zeros((16,), f32) for _ in range(_HC * n_out))

        def _compute(c, p):
            base = c * _G
            nv = nnb[c]
            mv = nm[c] if last_depth else None
            for g in range(_G):
                n_val = nv[g]

                def slot(k, carry):
                    r = g * _MAX_NB + k
                    out = []
                    for j in range(_HC):
                        s = pl.ds(j * 16, 16)
                        x = ra[p, r, s] + rb[p, r, s]
                        out.append(carry[j] + jnp.maximum(x, 0.0))
                    if last_depth:
                        for j in range(_HC):
                            s = pl.ds(j * 16, 16)
                            out.append(carry[_HC + j] + rna[p, r, s] * rnb[p, r, s])
                    return tuple(out)

                acc = lax.fori_loop(0, n_val, slot, zeros)
                for j in range(_HC):
                    stg[g, pl.ds(j * 16, 16)] = acc[j]
                if last_depth:
                    m = mv[g]
                    for j in range(_HC):
                        stg2[g, pl.ds(j * 16, 16)] = acc[_HC + j] * m
            row0 = wid * _PW + base
            pltpu.sync_copy(stg, nei_o.at[pl.ds(row0, _G)])
            if last_depth:
                pltpu.sync_copy(stg2, fnei_o.at[pl.ds(row0, _G)])

        _start(0, 0)

        def pair(jj, carry):
            c0 = 2 * jj
            _start(c0 + 1, 1)
            _wait(0)
            _compute(c0, 0)

            @pl.when(c0 + 2 < _NCH)
            def _():
                _start(c0 + 2, 0)

            _wait(1)
            _compute(c0 + 1, 1)
            return carry

        lax.fori_loop(0, _NCH // 2, pair, 0)

    mesh = plsc.VectorSubcoreMesh(core_axis_name="c", subcore_axis_name="s")
    return pl.kernel(body, mesh=mesh, out_type=out_type, scratch_types=scratch)


def kernel(input_atom, input_bond, atom_graph, bond_graph, num_nbs, node_mask,
           extra, W_af, W_na, W_nb, W_sa, W_u2, b_u2, W_u1, b_u1):
    f32 = jnp.float32
    fa = input_atom.shape[-1]
    fb = input_bond.shape[-1]

    xa = jnp.pad(input_atom.reshape(_RA, fa), ((0, 0), (0, _H - fa)))
    xb = jnp.pad(input_bond.reshape(_RB, fb), ((0, 0), (0, _H - fb)))
    waf = jnp.pad(W_af.astype(f32), ((0, _H - fa), (0, 0)))
    wnb = jnp.pad(W_nb.astype(f32), ((0, _H - fb), (0, 0)))
    wu2a = W_u2[:_H].astype(f32)
    wu2b = jnp.pad(W_u2[_H:].astype(f32), ((0, _H - fb), (0, 0)))
    w1a = W_u1[:_H].astype(f32)
    w1b = W_u1[_H:].astype(f32)
    bu2 = b_u2.reshape(1, _H).astype(f32)
    b1 = b_u1.reshape(1, _H).astype(f32)

    ag0 = atom_graph[..., 0].astype(jnp.int32).reshape(-1)
    ag1 = atom_graph[..., 1].astype(jnp.int32).reshape(-1)
    bg0 = bond_graph[..., 0].astype(jnp.int32).reshape(-1)
    bg1 = bond_graph[..., 1].astype(jnp.int32).reshape(-1)
    nnb = jnp.pad(num_nbs.astype(jnp.int32).reshape(_NW, _NCH, _G),
                  ((0, 0), (0, 0), (0, 16 - _G)))
    nm = jnp.pad(node_mask.astype(f32).reshape(_NW, _NCH, _G),
                 ((0, 0), (0, 0), (0, 16 - _G)))

    sc_mid = _sc_gather_reduce(False)
    sc_last = _sc_gather_reduce(True)

    af, t = _tc_atoms0(xa, waf, wu2a, bu2)
    bnb, bu2t = _tc_bonds(xb, wnb, wu2b)

    (nei,) = sc_mid(t, bu2t, ag0, ag1, bg0, bg1, nnb)
    af, t = _tc_update(False, af, nei, w1a, w1b, b1, wu2a, bu2)
    (nei,) = sc_mid(t, bu2t, ag0, ag1, bg0, bg1, nnb)
    af, t, na, sa = _tc_update(True, af, nei, w1a, w1b, b1, wu2a, bu2, W_na, W_sa)
    nei, fnei = sc_last(t, bu2t, na, bnb, ag0, ag1, bg0, bg1, nnb, nm)
    af, kern = _tc_final(af, nei, w1a, w1b, b1, fnei, sa)

    return (kern.reshape(_B, _N, _H), af.reshape(_B, _N, _H))


# speedup vs baseline: 26.7282x; 1.1656x over previous
import functools

import jax
import jax.numpy as jnp
from jax import lax
from jax.experimental import pallas as pl
from jax.experimental.pallas import tpu as pltpu
from jax.experimental.pallas import tpu_sc as plsc

_H = 128
_MAX_NB = 10
_DEPTH = 3
_B, _N, _NBOND = 128, 200, 400
_RA = _B * _N
_RB = _B * _NBOND
_NW = 32
_PW = _RA // _NW
_G = 8
_NCH = _PW // _G
_IPC = _G * _MAX_NB
_HC = _H // 16
_NFP = 5


_BRA = 1024
_BRB = 1024


def _rows(br):
    return pl.BlockSpec((br, _H), lambda i: (i, 0))


def _full(shape):
    return pl.BlockSpec(shape, lambda i: (0,) * len(shape))


def _pack_bf16(x):
    u = jax.lax.bitcast_convert_type(x.astype(jnp.bfloat16), jnp.uint16)
    u = u.astype(jnp.int32)
    return jax.lax.shift_left(u[:, _H // 2:], 16) | u[:, : _H // 2]


def _tc_atoms0_body(xa, waf, wu2a, bu2, af_o, t_o):
    af = jnp.dot(xa[...], waf[...], preferred_element_type=jnp.float32)
    af_o[...] = af
    t_o[...] = _pack_bf16(
        jnp.dot(af, wu2a[...], preferred_element_type=jnp.float32) + bu2[...])


def _tc_bonds_body(xb, wnb, wu2b, bnb_o, bu2_o):
    x = xb[...]
    bnb_o[...] = _pack_bf16(jnp.dot(x, wnb[...], preferred_element_type=jnp.float32))
    bu2_o[...] = _pack_bf16(jnp.dot(x, wu2b[...], preferred_element_type=jnp.float32))


def _tc_update_body(full_tables, af, nei, w1a, w1b, b1, wu2a, bu2, *rest):
    if full_tables:
        wna, wsa, af_o, t_o, na_o, sa_o = rest
    else:
        af_o, t_o = rest
    afn = jnp.dot(af[...], w1a[...], preferred_element_type=jnp.float32)
    afn = afn + jnp.dot(nei[...], w1b[...], preferred_element_type=jnp.float32)
    afn = jnp.maximum(afn + b1[...], 0.0)
    af_o[...] = afn
    t_o[...] = _pack_bf16(
        jnp.dot(afn, wu2a[...], preferred_element_type=jnp.float32) + bu2[...])
    if full_tables:
        na_o[...] = _pack_bf16(jnp.dot(afn, wna[...], preferred_element_type=jnp.float32))
        sa_o[...] = jnp.dot(afn, wsa[...], preferred_element_type=jnp.float32)


def _tc_final_body(af, nei, w1a, w1b, b1, fnei, sa, af_o, k_o):
    afn = jnp.dot(af[...], w1a[...], preferred_element_type=jnp.float32)
    afn = afn + jnp.dot(nei[...], w1b[...], preferred_element_type=jnp.float32)
    af_o[...] = jnp.maximum(afn + b1[...], 0.0)
    k_o[...] = fnei[...] * sa[...]


def _packed(br):
    return pl.BlockSpec((br, _H // 2), lambda i: (i, 0))


def _tc_atoms0(xa, waf, wu2a, bu2):
    sds = jax.ShapeDtypeStruct((_RA, _H), jnp.float32)
    pds = jax.ShapeDtypeStruct((_RA, _H // 2), jnp.int32)
    return pl.pallas_call(
        _tc_atoms0_body,
        grid=(_RA // _BRA,),
        in_specs=[pl.BlockSpec((_BRA, _H), lambda i: (i, 0)),
                  _full((_H, _H)), _full((_H, _H)), _full((1, _H))],
        out_specs=[_rows(_BRA), _packed(_BRA)],
        out_shape=[sds, pds],
    )(xa, waf, wu2a, bu2)


def _tc_bonds(xb, wnb, wu2b):
    pds = jax.ShapeDtypeStruct((_RB, _H // 2), jnp.int32)
    return pl.pallas_call(
        _tc_bonds_body,
        grid=(_RB // _BRB,),
        in_specs=[pl.BlockSpec((_BRB, _H), lambda i: (i, 0)),
                  _full((_H, _H)), _full((_H, _H))],
        out_specs=[_packed(_BRB), _packed(_BRB)],
        out_shape=[pds, pds],
    )(xb, wnb, wu2b)


def _tc_update(full_tables, af, nei, w1a, w1b, b1, wu2a, bu2, *weights):
    sds = jax.ShapeDtypeStruct((_RA, _H), jnp.float32)
    pds = jax.ShapeDtypeStruct((_RA, _H // 2), jnp.int32)
    out_shape = [sds, pds] + ([pds, sds] if full_tables else [])
    out_specs = [_rows(_BRA), _packed(_BRA)] + \
        ([_packed(_BRA), _rows(_BRA)] if full_tables else [])
    return pl.pallas_call(
        functools.partial(_tc_update_body, full_tables),
        grid=(_RA // _BRA,),
        in_specs=[_rows(_BRA), _rows(_BRA), _full((_H, _H)), _full((_H, _H)),
                  _full((1, _H)), _full((_H, _H)), _full((1, _H))]
                 + [_full((_H, _H))] * len(weights),
        out_specs=out_specs,
        out_shape=out_shape,
    )(af, nei, w1a, w1b, b1, wu2a, bu2, *weights)


def _tc_final(af, nei, w1a, w1b, b1, fnei, sa):
    sds = jax.ShapeDtypeStruct((_RA, _H), jnp.float32)
    return pl.pallas_call(
        _tc_final_body,
        grid=(_RA // _BRA,),
        in_specs=[_rows(_BRA), _rows(_BRA), _full((_H, _H)), _full((_H, _H)),
                  _full((1, _H)), _rows(_BRA), _rows(_BRA)],
        out_specs=[_rows(_BRA), _rows(_BRA)],
        out_shape=[sds, sds],
    )(af, nei, w1a, w1b, b1, fnei, sa)


def _sc_gather_reduce(last_depth):
    f32 = jnp.float32
    info = plsc.get_sparse_core_info()
    nc = info.num_cores

    n_out = 2 if last_depth else 1
    out_type = [jax.ShapeDtypeStruct((_RA, _H), f32)] * n_out
    n_tab = 4 if last_depth else 2
    scratch = (
        [pltpu.VMEM((_NCH * _IPC,), jnp.int32)] * 2
        + [pltpu.VMEM((_NCH * _IPC // _NFP,), jnp.int32)]
        + [pltpu.VMEM((_NCH, 16), jnp.int32)]
        + [pltpu.VMEM((2, _IPC, _H // 2), jnp.int32)] * n_tab
        + [pltpu.VMEM((_G, _H), f32)] * n_out
        + ([pltpu.VMEM((_NCH, 16), f32)] if last_depth else [])
        + [pltpu.SemaphoreType.DMA((2,))] * n_tab
    )

    def body(*refs):
        if last_depth:
            (ta, tb, tna, tnb, ag0, ag1, bg0, bg1, nnbh, nmh, nei_o, fnei_o,
             ia, ib, tmp, nnb, ra, rb, rna, rnb, stg, stg2, nm,
             sem_a, sem_b, sem_na, sem_nb) = refs
        else:
            (ta, tb, ag0, ag1, bg0, bg1, nnbh, nei_o,
             ia, ib, tmp, nnb, ra, rb, stg, sem_a, sem_b) = refs

        wid = lax.axis_index("s") * nc + lax.axis_index("c")

        n_idx = _NCH * _IPC
        npp = n_idx // _NFP

        def _flatten(dst, src0, src1, mul):
            pltpu.sync_copy(src0.at[pl.ds(wid * n_idx, n_idx)], dst)
            for q in range(_NFP):
                pltpu.sync_copy(src1.at[pl.ds(wid * n_idx + q * npp, npp)], tmp)

                def f(t, carry):
                    sd = pl.ds(q * npp + t * 16, 16)
                    st = pl.ds(t * 16, 16)
                    dst[sd] = dst[sd] * mul + tmp[st]
                    return carry
                lax.fori_loop(0, npp // 16, f, 0)

        _flatten(ia, ag0, ag1, _N)
        _flatten(ib, bg0, bg1, _NBOND)
        pltpu.sync_copy(nnbh.at[wid], nnb)
        if last_depth:
            pltpu.sync_copy(nmh.at[wid], nm)

        def _copies(c, p):
            sa = ia.at[pl.ds(c * _IPC, _IPC)]
            sb = ib.at[pl.ds(c * _IPC, _IPC)]
            cps = [pltpu.make_async_copy(ta.at[sa], ra.at[p], sem_a.at[p]),
                   pltpu.make_async_copy(tb.at[sb], rb.at[p], sem_b.at[p])]
            if last_depth:
                cps += [pltpu.make_async_copy(tna.at[sa], rna.at[p], sem_na.at[p]),
                        pltpu.make_async_copy(tnb.at[sb], rnb.at[p], sem_nb.at[p])]
            return cps

        def _start(c, p):
            for cp in _copies(c, p):
                cp.start()

        def _wait(p):
            for cp in _copies(0, p):
                cp.wait()

        zeros = tuple(jnp.zeros((16,), f32) for _ in range(_HC * n_out))
        hw = _HC // 2

        def _lo(v):
            return jax.lax.bitcast_convert_type(jax.lax.shift_left(v, 16), f32)

        def _hi(v):
            return jax.lax.bitcast_convert_type(
                jax.lax.bitwise_and(v, jnp.int32(-65536)), f32)

        def _compute(c, p):
            base = c * _G
            nv = nnb[c]
            mv = nm[c] if last_depth else None
            for g in range(_G):
                n_val = nv[g]

                def slot(k, carry):
                    r = g * _MAX_NB + k
                    out = list(carry)
                    for j in range(hw):
                        s = pl.ds(j * 16, 16)
                        va = ra[p, r, s]
                        vb = rb[p, r, s]
                        out[j] = out[j] + jnp.maximum(_lo(va) + _lo(vb), 0.0)
                        out[hw + j] = out[hw + j] + jnp.maximum(_hi(va) + _hi(vb), 0.0)
                    if last_depth:
                        for j in range(hw):
                            s = pl.ds(j * 16, 16)
                            vc = rna[p, r, s]
                            vd = rnb[p, r, s]
                            out[_HC + j] = out[_HC + j] + _lo(vc) * _lo(vd)
                            out[_HC + hw + j] = out[_HC + hw + j] + _hi(vc) * _hi(vd)
                    return tuple(out)

                acc = lax.fori_loop(0, n_val, slot, zeros)
                for j in range(_HC):
                    stg[g, pl.ds(j * 16, 16)] = acc[j]
                if last_depth:
                    m = mv[g]
                    for j in range(_HC):
                        stg2[g, pl.ds(j * 16, 16)] = acc[_HC + j] * m
            row0 = wid * _PW + base
            pltpu.sync_copy(stg, nei_o.at[pl.ds(row0, _G)])
            if last_depth:
                pltpu.sync_copy(stg2, fnei_o.at[pl.ds(row0, _G)])

        _start(0, 0)

        def pair(jj, carry):
            c0 = 2 * jj
            _start(c0 + 1, 1)
            _wait(0)
            _compute(c0, 0)

            @pl.when(c0 + 2 < _NCH)
            def _():
                _start(c0 + 2, 0)

            _wait(1)
            _compute(c0 + 1, 1)
            return carry

        lax.fori_loop(0, _NCH // 2, pair, 0)

    mesh = plsc.VectorSubcoreMesh(core_axis_name="c", subcore_axis_name="s")
    return pl.kernel(body, mesh=mesh, out_type=out_type, scratch_types=scratch,
                     compiler_params=pltpu.CompilerParams(use_tc_tiling_on_sc=False))


def kernel(input_atom, input_bond, atom_graph, bond_graph, num_nbs, node_mask,
           extra, W_af, W_na, W_nb, W_sa, W_u2, b_u2, W_u1, b_u1):
    f32 = jnp.float32
    fa = input_atom.shape[-1]
    fb = input_bond.shape[-1]

    xa = jnp.pad(input_atom.reshape(_RA, fa), ((0, 0), (0, _H - fa)))
    xb = jnp.pad(input_bond.reshape(_RB, fb), ((0, 0), (0, _H - fb)))
    waf = jnp.pad(W_af.astype(f32), ((0, _H - fa), (0, 0)))
    wnb = jnp.pad(W_nb.astype(f32), ((0, _H - fb), (0, 0)))
    wu2a = W_u2[:_H].astype(f32)
    wu2b = jnp.pad(W_u2[_H:].astype(f32), ((0, _H - fb), (0, 0)))
    w1a = W_u1[:_H].astype(f32)
    w1b = W_u1[_H:].astype(f32)
    bu2 = b_u2.reshape(1, _H).astype(f32)
    b1 = b_u1.reshape(1, _H).astype(f32)

    ag0 = atom_graph[..., 0].astype(jnp.int32).reshape(-1)
    ag1 = atom_graph[..., 1].astype(jnp.int32).reshape(-1)
    bg0 = bond_graph[..., 0].astype(jnp.int32).reshape(-1)
    bg1 = bond_graph[..., 1].astype(jnp.int32).reshape(-1)
    nnb = jnp.pad(num_nbs.astype(jnp.int32).reshape(_NW, _NCH, _G),
                  ((0, 0), (0, 0), (0, 16 - _G)))
    nm = jnp.pad(node_mask.astype(f32).reshape(_NW, _NCH, _G),
                 ((0, 0), (0, 0), (0, 16 - _G)))

    sc_mid = _sc_gather_reduce(False)
    sc_last = _sc_gather_reduce(True)

    af, t = _tc_atoms0(xa, waf, wu2a, bu2)
    bnb, bu2t = _tc_bonds(xb, wnb, wu2b)

    (nei,) = sc_mid(t, bu2t, ag0, ag1, bg0, bg1, nnb)
    af, t = _tc_update(False, af, nei, w1a, w1b, b1, wu2a, bu2)
    (nei,) = sc_mid(t, bu2t, ag0, ag1, bg0, bg1, nnb)
    af, t, na, sa = _tc_update(True, af, nei, w1a, w1b, b1, wu2a, bu2, W_na, W_sa)
    nei, fnei = sc_last(t, bu2t, na, bnb, ag0, ag1, bg0, bg1, nnb, nm)
    af, kern = _tc_final(af, nei, w1a, w1b, b1, fnei, sa)

    return (kern.reshape(_B, _N, _H), af.reshape(_B, _N, _H))
